# baseline (speedup 1.0000x reference)
import functools

import jax
import jax.numpy as jnp
from jax import lax
from jax.experimental import pallas as pl
from jax.experimental.pallas import tpu as pltpu
from jax.experimental.pallas import tpu_sc as plsc

N = 10000
E = 320000
D = 128
NG = 512

NC = 2
NS = 16
NW = NC * NS
EPT = E // NW
CH = 40
NF = EPT // CH
NCT = NF
NP = 10112
RPT = NP // NS
RCH = 128
NGP = NG + 16
NIDC = 5

_LANES = 16


def _zero_vmem_rows(ref, nrows, ncols):
    def body(r, _):
        for j in range(ncols // _LANES):
            ref[r, pl.ds(j * _LANES, _LANES)] = jnp.zeros((_LANES,), jnp.float32)
        return 0
    lax.fori_loop(0, nrows, body, 0)


def _edge_aggregate(node_hidden, idx4, edge_hidden):
    mesh = plsc.VectorSubcoreMesh(
        core_axis_name="c", subcore_axis_name="s",
        num_cores=NC, num_subcores=NS)

    @functools.partial(
        pl.kernel,
        out_type=jax.ShapeDtypeStruct((NC, NP, D), jnp.float32),
        mesh=mesh,
        scratch_types=[
            [pltpu.VMEM((CH,), jnp.int32) for _ in range(4)],
            [pltpu.VMEM((CH,), jnp.int32) for _ in range(2)],
            [pltpu.VMEM((CH, D), jnp.float32) for _ in range(2)],
            [pltpu.VMEM((CH, D), jnp.float32) for _ in range(2)],
            [pltpu.VMEM((CH, D), jnp.float32) for _ in range(2)],
            pltpu.VMEM_SHARED((NP, D), jnp.float32),
            [pltpu.SemaphoreType.DMA for _ in range(4)],
            [pltpu.SemaphoreType.DMA for _ in range(2)],
            [pltpu.SemaphoreType.DMA for _ in range(2)],
            [pltpu.SemaphoreType.DMA for _ in range(2)],
            [pltpu.SemaphoreType.DMA for _ in range(2)],
        ],
    )
    def k(nh_hbm, idx_hbm, eh_hbm, out_hbm,
          sidx, didx, rows, erows, mbuf,
          acc, isem, dsem, gsem, esem, ssem):
        cid = lax.axis_index("c")
        sid = lax.axis_index("s")
        wid = cid * NS + sid

        _zero_vmem_rows(mbuf[0], CH, D)
        row0 = sid * RPT
        for kk in range(RPT // CH):
            pltpu.sync_copy(mbuf[0], acc.at[pl.ds(row0 + kk * CH, CH)])
        pltpu.sync_copy(mbuf[0].at[pl.ds(0, RPT - (RPT // CH) * CH)],
                        acc.at[pl.ds(row0 + (RPT // CH) * CH,
                                     RPT - (RPT // CH) * CH)])
        plsc.subcore_barrier()

        ebase = wid * EPT

        def issue_sidx(c, m):
            pltpu.async_copy(idx_hbm.at[wid, c, 0], sidx[m], isem[m])

        def wait_sidx(c, m):
            pltpu.make_async_copy(
                idx_hbm.at[wid, c, 0], sidx[m], isem[m]).wait()

        def issue_gather(c, b, m):
            pltpu.async_copy(nh_hbm.at[sidx[m]], rows[b], gsem[b])
            pltpu.async_copy(eh_hbm.at[pl.ds(ebase + c * CH, CH)],
                             erows[b], esem[b])

        def wait_gather(c, b, m):
            pltpu.make_async_copy(
                nh_hbm.at[sidx[m]], rows[b], gsem[b]).wait()
            pltpu.make_async_copy(
                eh_hbm.at[pl.ds(ebase + c * CH, CH)], erows[b],
                esem[b]).wait()

        def compute(b, nrows):
            def rbody(r, _):
                for j in range(D // _LANES):
                    sl = pl.ds(j * _LANES, _LANES)
                    v = rows[b][r, sl] + erows[b][r, sl]
                    mbuf[b][r, sl] = jnp.maximum(v, 0.0)
                return 0
            lax.fori_loop(0, nrows, rbody, 0)

        def body(c, m, b, first, last):
            wait_gather(c, b, m)

            if not last:
                @pl.when(c + 4 < NF)
                def _prefetch_sidx():
                    issue_sidx(c + 4, m)

            if first:
                @pl.when(c >= 2)
                def _wait_scatter():
                    pltpu.make_async_copy(
                        mbuf[b], acc.at[didx[b]], ssem[b]).wait()
            else:
                pltpu.make_async_copy(
                    mbuf[b], acc.at[didx[b]], ssem[b]).wait()

            pltpu.async_copy(idx_hbm.at[wid, c, 1], didx[b], dsem[b])
            compute(b, CH)

            if not last:
                @pl.when(c + 2 < NF)
                def _issue_next():
                    wait_sidx(c + 2, (m + 2) % 4)
                    issue_gather(c + 2, b, (m + 2) % 4)

            pltpu.make_async_copy(
                idx_hbm.at[wid, c, 1], didx[b], dsem[b]).wait()
            pltpu.async_copy(mbuf[b], acc.at[didx[b]], ssem[b], add=True)

        for c0 in range(4):
            issue_sidx(c0, c0)
        wait_sidx(0, 0)
        wait_sidx(1, 1)
        issue_gather(0, 0, 0)
        issue_gather(1, 1, 1)

        MAIN = NF - 2

        @pl.loop(0, MAIN, step=4)
        def quad(i):
            for q in range(4):
                body(i + q, q, q % 2, first=(q < 2), last=False)

        body(MAIN, MAIN % 4, 0, first=False, last=True)
        body(MAIN + 1, (MAIN + 1) % 4, 1, first=False, last=True)

        pltpu.make_async_copy(mbuf[0], acc.at[didx[0]], ssem[0]).wait()
        pltpu.make_async_copy(mbuf[1], acc.at[didx[1]], ssem[1]).wait()

        plsc.subcore_barrier()

        for kk in range(RPT // RCH):
            r0 = row0 + kk * RCH
            pltpu.sync_copy(acc.at[pl.ds(r0, RCH)],
                            out_hbm.at[cid, pl.ds(r0, RCH)])
        REM = RPT - (RPT // RCH) * RCH
        if REM:
            r0 = row0 + (RPT // RCH) * RCH
            pltpu.sync_copy(acc.at[pl.ds(r0, REM)],
                            out_hbm.at[cid, pl.ds(r0, REM)])

    return k(node_hidden, idx4, edge_hidden)


def _count_kernel(nid2):
    BN = 1000
    NB = N // BN

    def body(nid_ref, out_ref):
        i = pl.program_id(0)

        @pl.when(i == 0)
        def _init():
            out_ref[...] = jnp.zeros_like(out_ref)

        ids = nid_ref[...]
        iota = lax.broadcasted_iota(jnp.int32, (BN, NG), 1)
        onehot = (ids == iota).astype(jnp.float32)
        out_ref[...] += jnp.sum(onehot, axis=0, keepdims=True)

    return pl.pallas_call(
        body,
        grid=(NB,),
        in_specs=[pl.BlockSpec((BN, 1), lambda i: (i, 0))],
        out_specs=pl.BlockSpec((1, NG), lambda i: (0, 0)),
        out_shape=jax.ShapeDtypeStruct((1, NG), jnp.float32),
    )(nid2)


def _node_kernel(node_hidden, partials, nid2, counts, W1, b1, W2, b2,
                 ln_gamma, ln_beta):
    BN = 1000
    NB = N // BN

    def body(nh_ref, part_ref, nid_ref, cnt_ref, w1_ref, b1_ref, w2_ref,
             b2_ref, g_ref, beta_ref, out_ref):
        nh = nh_ref[...]
        h = nh + part_ref[0] + part_ref[1]
        h1 = jnp.maximum(
            jnp.dot(h, w1_ref[...], preferred_element_type=jnp.float32)
            + b1_ref[...], 0.0)
        h2 = (jnp.dot(h1, w2_ref[...], preferred_element_type=jnp.float32)
              + b2_ref[...])
        mean = jnp.mean(h2, axis=-1, keepdims=True)
        var = jnp.mean((h2 - mean) ** 2, axis=-1, keepdims=True)
        ln = (h2 - mean) * lax.rsqrt(var + 1e-5) * g_ref[...] + beta_ref[...]
        ids = nid_ref[...]
        iota = lax.broadcasted_iota(jnp.int32, (BN, NG), 1)
        onehot = (ids == iota).astype(jnp.float32)
        cnt = jnp.dot(onehot, cnt_ref[...],
                      preferred_element_type=jnp.float32,
                      precision=lax.Precision.HIGHEST)
        out_ref[...] = jnp.maximum(ln * lax.rsqrt(cnt), 0.0) + nh

    return pl.pallas_call(
        body,
        grid=(NB,),
        in_specs=[
            pl.BlockSpec((BN, D), lambda i: (i, 0)),
            pl.BlockSpec((NC, BN, D), lambda i: (0, i, 0)),
            pl.BlockSpec((BN, 1), lambda i: (i, 0)),
            pl.BlockSpec((NG, 1), lambda i: (0, 0)),
            pl.BlockSpec((D, 2 * D), lambda i: (0, 0)),
            pl.BlockSpec((1, 2 * D), lambda i: (0, 0)),
            pl.BlockSpec((2 * D, D), lambda i: (0, 0)),
            pl.BlockSpec((1, D), lambda i: (0, 0)),
            pl.BlockSpec((1, D), lambda i: (0, 0)),
            pl.BlockSpec((1, D), lambda i: (0, 0)),
        ],
        out_specs=pl.BlockSpec((BN, D), lambda i: (i, 0)),
        out_shape=jax.ShapeDtypeStruct((N, D), jnp.float32),
    )(node_hidden, partials, nid2, counts, W1, b1[None, :], W2, b2[None, :],
      ln_gamma[None, :], ln_beta[None, :])


def kernel(node_hidden, edge_index, edge_hidden, node_id, edge_id,
           W1, b1, W2, b2, ln_gamma, ln_beta):
    srcp = edge_index[0].astype(jnp.int32).reshape(NW, NCT, CH)
    dstp = edge_index[1].astype(jnp.int32).reshape(NW, NCT, CH)
    idx4 = jnp.stack([srcp, dstp], axis=2)
    nid = node_id.astype(jnp.int32)
    partials = _edge_aggregate(node_hidden, idx4, edge_hidden)
    nid2 = nid.reshape(N, 1)
    counts = _count_kernel(nid2).reshape(NG, 1)
    return _node_kernel(node_hidden, partials, nid2, counts,
                        W1, b1, W2, b2, ln_gamma, ln_beta)


# speedup vs baseline: 1.0975x; 1.0975x over previous
import functools

import jax
import jax.numpy as jnp
from jax import lax
from jax.experimental import pallas as pl
from jax.experimental.pallas import tpu as pltpu
from jax.experimental.pallas import tpu_sc as plsc

N = 10000
E = 320000
D = 128
NG = 512

NC = 2
NS = 16
NW = NC * NS
EPT = E // NW
CH = 40
NF = EPT // CH
NCT = NF
NP = 10240
RPT = NP // NS
RCH = 128
NGP = NG + 16
NIDC = 5

_LANES = 16


def _zero_vmem_rows(ref, nrows, ncols):
    def body(r, _):
        for j in range(ncols // _LANES):
            ref[r, pl.ds(j * _LANES, _LANES)] = jnp.zeros((_LANES,), jnp.float32)
        return 0
    lax.fori_loop(0, nrows, body, 0)


def _edge_aggregate(node_hidden, src3, dst3, edge_hidden):
    mesh = plsc.VectorSubcoreMesh(
        core_axis_name="c", subcore_axis_name="s",
        num_cores=NC, num_subcores=NS)

    @functools.partial(
        pl.kernel,
        out_type=jax.ShapeDtypeStruct((NC, NP, D), jnp.float32),
        mesh=mesh,
        scratch_types=[
            [pltpu.VMEM((CH,), jnp.int32) for _ in range(4)],
            [pltpu.VMEM((CH,), jnp.int32) for _ in range(2)],
            [pltpu.VMEM((CH, D), jnp.float32) for _ in range(2)],
            [pltpu.VMEM((CH, D), jnp.float32) for _ in range(2)],
            [pltpu.VMEM((CH, D), jnp.float32) for _ in range(2)],
            pltpu.VMEM_SHARED((NP, D), jnp.float32),
            [pltpu.SemaphoreType.DMA for _ in range(4)],
            [pltpu.SemaphoreType.DMA for _ in range(2)],
            [pltpu.SemaphoreType.DMA for _ in range(2)],
            [pltpu.SemaphoreType.DMA for _ in range(2)],
            [pltpu.SemaphoreType.DMA for _ in range(2)],
        ],
    )
    def k(nh_hbm, src_hbm, dst_hbm, eh_hbm, out_hbm,
          sidx, didx, rows, erows, mbuf,
          acc, isem, dsem, gsem, esem, ssem):
        cid = lax.axis_index("c")
        sid = lax.axis_index("s")
        wid = cid * NS + sid

        _zero_vmem_rows(mbuf[0], CH, D)
        row0 = sid * RPT
        for kk in range(RPT // CH):
            pltpu.sync_copy(mbuf[0], acc.at[pl.ds(row0 + kk * CH, CH)])
        if RPT % CH:
            pltpu.sync_copy(mbuf[0].at[pl.ds(0, RPT % CH)],
                            acc.at[pl.ds(row0 + (RPT // CH) * CH, RPT % CH)])
        plsc.subcore_barrier()

        ebase = wid * EPT

        def issue_sidx(c, m):
            pltpu.async_copy(src_hbm.at[wid, c], sidx[m], isem[m])

        def wait_sidx(c, m):
            pltpu.make_async_copy(
                src_hbm.at[wid, c], sidx[m], isem[m]).wait()

        def issue_gather(c, b, m):
            pltpu.async_copy(nh_hbm.at[sidx[m]], rows[b], gsem[b])
            pltpu.async_copy(eh_hbm.at[pl.ds(ebase + c * CH, CH)],
                             erows[b], esem[b])

        def wait_gather(c, b, m):
            pltpu.make_async_copy(
                nh_hbm.at[sidx[m]], rows[b], gsem[b]).wait()
            pltpu.make_async_copy(
                eh_hbm.at[pl.ds(ebase + c * CH, CH)], erows[b],
                esem[b]).wait()

        def compute(b, nrows):
            def rbody(r, _):
                for j in range(D // _LANES):
                    sl = pl.ds(j * _LANES, _LANES)
                    v = rows[b][r, sl] + erows[b][r, sl]
                    mbuf[b][r, sl] = jnp.maximum(v, 0.0)
                return 0
            lax.fori_loop(0, nrows, rbody, 0)

        def body(c, m, b, first, last):
            wait_gather(c, b, m)

            if not last:
                @pl.when(c + 4 < NF)
                def _prefetch_sidx():
                    issue_sidx(c + 4, m)

            if first:
                @pl.when(c >= 2)
                def _wait_scatter():
                    pltpu.make_async_copy(
                        mbuf[b], acc.at[didx[b]], ssem[b]).wait()
            else:
                pltpu.make_async_copy(
                    mbuf[b], acc.at[didx[b]], ssem[b]).wait()

            pltpu.async_copy(dst_hbm.at[wid, c], didx[b], dsem[b])
            compute(b, CH)

            if not last:
                @pl.when(c + 2 < NF)
                def _issue_next():
                    wait_sidx(c + 2, (m + 2) % 4)
                    issue_gather(c + 2, b, (m + 2) % 4)

            pltpu.make_async_copy(
                dst_hbm.at[wid, c], didx[b], dsem[b]).wait()
            pltpu.async_copy(mbuf[b], acc.at[didx[b]], ssem[b], add=True)

        for c0 in range(4):
            issue_sidx(c0, c0)
        wait_sidx(0, 0)
        wait_sidx(1, 1)
        issue_gather(0, 0, 0)
        issue_gather(1, 1, 1)

        MAIN = NF - 2

        @pl.loop(0, MAIN, step=4)
        def quad(i):
            for q in range(4):
                body(i + q, q, q % 2, first=(q < 2), last=False)

        body(MAIN, MAIN % 4, 0, first=False, last=True)
        body(MAIN + 1, (MAIN + 1) % 4, 1, first=False, last=True)

        pltpu.make_async_copy(mbuf[0], acc.at[didx[0]], ssem[0]).wait()
        pltpu.make_async_copy(mbuf[1], acc.at[didx[1]], ssem[1]).wait()

        plsc.subcore_barrier()

        for kk in range(RPT // RCH):
            r0 = row0 + kk * RCH
            pltpu.sync_copy(acc.at[pl.ds(r0, RCH)],
                            out_hbm.at[cid, pl.ds(r0, RCH)])
        REM = RPT - (RPT // RCH) * RCH
        if REM:
            r0 = row0 + (RPT // RCH) * RCH
            pltpu.sync_copy(acc.at[pl.ds(r0, REM)],
                            out_hbm.at[cid, pl.ds(r0, REM)])

    return k(node_hidden, src3, dst3, edge_hidden)


def _count_kernel(nid2):
    BN = 1000
    NB = N // BN

    def body(nid_ref, out_ref):
        i = pl.program_id(0)

        @pl.when(i == 0)
        def _init():
            out_ref[...] = jnp.zeros_like(out_ref)

        ids = nid_ref[...]
        iota = lax.broadcasted_iota(jnp.int32, (BN, NG), 1)
        onehot = (ids == iota).astype(jnp.float32)
        out_ref[...] += jnp.sum(onehot, axis=0, keepdims=True)

    return pl.pallas_call(
        body,
        grid=(NB,),
        in_specs=[pl.BlockSpec((BN, 1), lambda i: (i, 0))],
        out_specs=pl.BlockSpec((1, NG), lambda i: (0, 0)),
        out_shape=jax.ShapeDtypeStruct((1, NG), jnp.float32),
    )(nid2)


def _node_kernel(node_hidden, partials, nid2, counts, W1, b1, W2, b2,
                 ln_gamma, ln_beta):
    BN = 1000
    NB = N // BN

    def body(nh_ref, part_ref, nid_ref, cnt_ref, w1_ref, b1_ref, w2_ref,
             b2_ref, g_ref, beta_ref, out_ref):
        nh = nh_ref[...]
        h = nh + part_ref[0] + part_ref[1]
        h1 = jnp.maximum(
            jnp.dot(h, w1_ref[...], preferred_element_type=jnp.float32)
            + b1_ref[...], 0.0)
        h2 = (jnp.dot(h1, w2_ref[...], preferred_element_type=jnp.float32)
              + b2_ref[...])
        mean = jnp.mean(h2, axis=-1, keepdims=True)
        var = jnp.mean((h2 - mean) ** 2, axis=-1, keepdims=True)
        ln = (h2 - mean) * lax.rsqrt(var + 1e-5) * g_ref[...] + beta_ref[...]
        ids = nid_ref[...]
        iota = lax.broadcasted_iota(jnp.int32, (BN, NG), 1)
        onehot = (ids == iota).astype(jnp.float32)
        cnt = jnp.dot(onehot, cnt_ref[...],
                      preferred_element_type=jnp.float32,
                      precision=lax.Precision.HIGHEST)
        out_ref[...] = jnp.maximum(ln * lax.rsqrt(cnt), 0.0) + nh

    return pl.pallas_call(
        body,
        grid=(NB,),
        in_specs=[
            pl.BlockSpec((BN, D), lambda i: (i, 0)),
            pl.BlockSpec((NC, BN, D), lambda i: (0, i, 0)),
            pl.BlockSpec((BN, 1), lambda i: (i, 0)),
            pl.BlockSpec((NG, 1), lambda i: (0, 0)),
            pl.BlockSpec((D, 2 * D), lambda i: (0, 0)),
            pl.BlockSpec((1, 2 * D), lambda i: (0, 0)),
            pl.BlockSpec((2 * D, D), lambda i: (0, 0)),
            pl.BlockSpec((1, D), lambda i: (0, 0)),
            pl.BlockSpec((1, D), lambda i: (0, 0)),
            pl.BlockSpec((1, D), lambda i: (0, 0)),
        ],
        out_specs=pl.BlockSpec((BN, D), lambda i: (i, 0)),
        out_shape=jax.ShapeDtypeStruct((N, D), jnp.float32),
    )(node_hidden, partials, nid2, counts, W1, b1[None, :], W2, b2[None, :],
      ln_gamma[None, :], ln_beta[None, :])


def kernel(node_hidden, edge_index, edge_hidden, node_id, edge_id,
           W1, b1, W2, b2, ln_gamma, ln_beta):
    src3 = edge_index[0].astype(jnp.int32).reshape(NW, NCT, CH)
    dst3 = edge_index[1].astype(jnp.int32).reshape(NW, NCT, CH)
    nid = node_id.astype(jnp.int32)
    partials = _edge_aggregate(node_hidden, src3, dst3, edge_hidden)
    nid2 = nid.reshape(N, 1)
    counts = _count_kernel(nid2).reshape(NG, 1)
    return _node_kernel(node_hidden, partials, nid2, counts,
                        W1, b1, W2, b2, ln_gamma, ln_beta)


# speedup vs baseline: 1.2006x; 1.0939x over previous
import functools

import jax
import jax.numpy as jnp
from jax import lax
from jax.experimental import pallas as pl
from jax.experimental.pallas import tpu as pltpu
from jax.experimental.pallas import tpu_sc as plsc

N = 10000
E = 320000
D = 128
NG = 512

NC = 2
NS = 16
NW = NC * NS
EPT = E // NW
CH = 64
NF = EPT // CH
TAIL = EPT - NF * CH
NP = 10240
RPT = NP // NS
RCH = 128
NGP = NG + 16
NIDC = 5

_LANES = 16


def _zero_vmem_rows(ref, nrows, ncols):
    def body(r, _):
        for j in range(ncols // _LANES):
            ref[r, pl.ds(j * _LANES, _LANES)] = jnp.zeros((_LANES,), jnp.float32)
        return 0
    lax.fori_loop(0, nrows, body, 0)


def _edge_aggregate(node_hidden, src1, dst1, edge_hidden):
    mesh = plsc.VectorSubcoreMesh(
        core_axis_name="c", subcore_axis_name="s",
        num_cores=NC, num_subcores=NS)

    @functools.partial(
        pl.kernel,
        out_type=jax.ShapeDtypeStruct((NC, NP, D), jnp.float32),
        mesh=mesh,
        scratch_types=[
            [pltpu.VMEM((CH,), jnp.int32) for _ in range(4)],
            [pltpu.VMEM((CH,), jnp.int32) for _ in range(2)],
            [pltpu.VMEM((CH, D), jnp.float32) for _ in range(2)],
            [pltpu.VMEM((CH, D), jnp.float32) for _ in range(2)],
            pltpu.VMEM((TAIL,), jnp.int32),
            pltpu.VMEM((TAIL,), jnp.int32),
            pltpu.VMEM_SHARED((NP, D), jnp.float32),
            [pltpu.SemaphoreType.DMA for _ in range(4)],
            [pltpu.SemaphoreType.DMA for _ in range(2)],
            [pltpu.SemaphoreType.DMA for _ in range(2)],
            [pltpu.SemaphoreType.DMA for _ in range(2)],
            [pltpu.SemaphoreType.DMA for _ in range(2)],
        ],
    )
    def k(nh_hbm, src_hbm, dst_hbm, eh_hbm, out_hbm,
          sidx, didx, rows, erows, sidx_t, didx_t,
          acc, isem, dsem, gsem, esem, ssem):
        cid = lax.axis_index("c")
        sid = lax.axis_index("s")
        wid = cid * NS + sid

        _zero_vmem_rows(erows[0], CH, D)
        row0 = sid * RPT
        for kk in range(RPT // CH):
            pltpu.sync_copy(erows[0], acc.at[pl.ds(row0 + kk * CH, CH)])
        if RPT % CH:
            pltpu.sync_copy(erows[0].at[pl.ds(0, RPT % CH)],
                            acc.at[pl.ds(row0 + (RPT // CH) * CH, RPT % CH)])
        plsc.subcore_barrier()

        ebase = wid * EPT

        def issue_sidx(c, m):
            pltpu.async_copy(src_hbm.at[pl.ds(ebase + c * CH, CH)],
                             sidx[m], isem[m])

        def wait_sidx(c, m):
            pltpu.make_async_copy(
                src_hbm.at[pl.ds(ebase + c * CH, CH)], sidx[m],
                isem[m]).wait()

        def compute(b, nrows):
            def rbody(r, _):
                for j in range(D // _LANES):
                    sl = pl.ds(j * _LANES, _LANES)
                    v = rows[b][r, sl] + erows[b][r, sl]
                    erows[b][r, sl] = jnp.maximum(v, 0.0)
                return 0
            lax.fori_loop(0, nrows, rbody, 0)

        def body(c, m, b, prefetch_ok, issue_ok, static_tail):
            pltpu.make_async_copy(
                nh_hbm.at[sidx[m]], rows[b], gsem[b]).wait()
            pltpu.make_async_copy(
                eh_hbm.at[pl.ds(ebase + c * CH, CH)], erows[b],
                esem[b]).wait()

            def _prefetch():
                issue_sidx(c + 4, m)
            if prefetch_ok is None:
                pl.when(c + 4 < NF)(_prefetch)
            elif prefetch_ok:
                _prefetch()

            pltpu.async_copy(dst_hbm.at[pl.ds(ebase + c * CH, CH)],
                             didx[b], dsem[b])

            compute(b, CH)

            def _issue_gather():
                wait_sidx(c + 2, (m + 2) % 4)
                pltpu.async_copy(nh_hbm.at[sidx[(m + 2) % 4]], rows[b],
                                 gsem[b])
            if issue_ok is None:
                pl.when(c + 2 < NF)(_issue_gather)
            elif issue_ok:
                _issue_gather()

            pltpu.make_async_copy(
                dst_hbm.at[pl.ds(ebase + c * CH, CH)], didx[b],
                dsem[b]).wait()
            pltpu.async_copy(erows[b], acc.at[didx[b]], ssem[b], add=True)
            pltpu.make_async_copy(
                erows[b], acc.at[didx[b]], ssem[b]).wait()

            def _issue_edge():
                pltpu.async_copy(eh_hbm.at[pl.ds(ebase + (c + 2) * CH, CH)],
                                 erows[b], esem[b])
            if issue_ok is None:
                pl.when(c + 2 < NF)(_issue_edge)
            elif issue_ok:
                _issue_edge()

        for c0 in range(4):
            issue_sidx(c0, c0)
        for c0 in range(2):
            wait_sidx(c0, c0)
            pltpu.async_copy(nh_hbm.at[sidx[c0]], rows[c0], gsem[c0])
            pltpu.async_copy(eh_hbm.at[pl.ds(ebase + c0 * CH, CH)],
                             erows[c0], esem[c0])

        MAIN = (NF - 2) // 4 * 4

        @pl.loop(0, MAIN, step=4)
        def quad(i):
            for q in range(4):
                body(i + q, q, q % 2, None, None, False)

        for c in range(MAIN, NF):
            body(c, c % 4, c % 2, c + 4 < NF, c + 2 < NF, False)

        if TAIL:
            pltpu.sync_copy(src_hbm.at[pl.ds(ebase + NF * CH, TAIL)], sidx_t)
            pltpu.sync_copy(dst_hbm.at[pl.ds(ebase + NF * CH, TAIL)], didx_t)
            pltpu.async_copy(nh_hbm.at[sidx_t], rows[0].at[pl.ds(0, TAIL)],
                             gsem[0]).wait()
            pltpu.sync_copy(eh_hbm.at[pl.ds(ebase + NF * CH, TAIL)],
                            erows[0].at[pl.ds(0, TAIL)])
            compute(0, TAIL)
            pltpu.sync_copy(erows[0].at[pl.ds(0, TAIL)], acc.at[didx_t],
                            add=True)

        plsc.subcore_barrier()

        for kk in range(RPT // RCH):
            r0 = row0 + kk * RCH
            pltpu.sync_copy(acc.at[pl.ds(r0, RCH)],
                            out_hbm.at[cid, pl.ds(r0, RCH)])
        REM = RPT - (RPT // RCH) * RCH
        if REM:
            r0 = row0 + (RPT // RCH) * RCH
            pltpu.sync_copy(acc.at[pl.ds(r0, REM)],
                            out_hbm.at[cid, pl.ds(r0, REM)])

    return k(node_hidden, src1, dst1, edge_hidden)


def _count_kernel(nid2):
    BN = 1000
    NB = N // BN

    def body(nid_ref, out_ref):
        i = pl.program_id(0)

        @pl.when(i == 0)
        def _init():
            out_ref[...] = jnp.zeros_like(out_ref)

        ids = nid_ref[...]
        iota = lax.broadcasted_iota(jnp.int32, (BN, NG), 1)
        onehot = (ids == iota).astype(jnp.float32)
        out_ref[...] += jnp.sum(onehot, axis=0, keepdims=True)

    return pl.pallas_call(
        body,
        grid=(NB,),
        in_specs=[pl.BlockSpec((BN, 1), lambda i: (i, 0))],
        out_specs=pl.BlockSpec((1, NG), lambda i: (0, 0)),
        out_shape=jax.ShapeDtypeStruct((1, NG), jnp.float32),
    )(nid2)


def _node_kernel(node_hidden, partials, nid2, counts, W1, b1, W2, b2,
                 ln_gamma, ln_beta):
    BN = 1000
    NB = N // BN

    def body(nh_ref, part_ref, nid_ref, cnt_ref, w1_ref, b1_ref, w2_ref,
             b2_ref, g_ref, beta_ref, out_ref):
        nh = nh_ref[...]
        h = nh + part_ref[0] + part_ref[1]
        h1 = jnp.maximum(
            jnp.dot(h, w1_ref[...], preferred_element_type=jnp.float32)
            + b1_ref[...], 0.0)
        h2 = (jnp.dot(h1, w2_ref[...], preferred_element_type=jnp.float32)
              + b2_ref[...])
        mean = jnp.mean(h2, axis=-1, keepdims=True)
        var = jnp.mean((h2 - mean) ** 2, axis=-1, keepdims=True)
        ln = (h2 - mean) * lax.rsqrt(var + 1e-5) * g_ref[...] + beta_ref[...]
        ids = nid_ref[...]
        iota = lax.broadcasted_iota(jnp.int32, (BN, NG), 1)
        onehot = (ids == iota).astype(jnp.float32)
        cnt = jnp.dot(onehot, cnt_ref[...],
                      preferred_element_type=jnp.float32,
                      precision=lax.Precision.HIGHEST)
        out_ref[...] = jnp.maximum(ln * lax.rsqrt(cnt), 0.0) + nh

    return pl.pallas_call(
        body,
        grid=(NB,),
        in_specs=[
            pl.BlockSpec((BN, D), lambda i: (i, 0)),
            pl.BlockSpec((NC, BN, D), lambda i: (0, i, 0)),
            pl.BlockSpec((BN, 1), lambda i: (i, 0)),
            pl.BlockSpec((NG, 1), lambda i: (0, 0)),
            pl.BlockSpec((D, 2 * D), lambda i: (0, 0)),
            pl.BlockSpec((1, 2 * D), lambda i: (0, 0)),
            pl.BlockSpec((2 * D, D), lambda i: (0, 0)),
            pl.BlockSpec((1, D), lambda i: (0, 0)),
            pl.BlockSpec((1, D), lambda i: (0, 0)),
            pl.BlockSpec((1, D), lambda i: (0, 0)),
        ],
        out_specs=pl.BlockSpec((BN, D), lambda i: (i, 0)),
        out_shape=jax.ShapeDtypeStruct((N, D), jnp.float32),
    )(node_hidden, partials, nid2, counts, W1, b1[None, :], W2, b2[None, :],
      ln_gamma[None, :], ln_beta[None, :])


def kernel(node_hidden, edge_index, edge_hidden, node_id, edge_id,
           W1, b1, W2, b2, ln_gamma, ln_beta):
    src1 = edge_index[0].astype(jnp.int32)
    dst1 = edge_index[1].astype(jnp.int32)
    nid = node_id.astype(jnp.int32)
    partials = _edge_aggregate(node_hidden, src1, dst1, edge_hidden)
    nid2 = nid.reshape(N, 1)
    counts = _count_kernel(nid2).reshape(NG, 1)
    return _node_kernel(node_hidden, partials, nid2, counts,
                        W1, b1, W2, b2, ln_gamma, ln_beta)


# speedup vs baseline: 1.2179x; 1.0144x over previous
import functools

import jax
import jax.numpy as jnp
from jax import lax
from jax.experimental import pallas as pl
from jax.experimental.pallas import tpu as pltpu
from jax.experimental.pallas import tpu_sc as plsc

N = 10000
E = 320000
D = 128
NG = 512

NC = 2
NS = 16
NW = NC * NS
EPT = E // NW
CH = 64
NF = EPT // CH
TAIL = EPT - NF * CH
NP = 10240
RPT = NP // NS
RCH = 128
NGP = NG + 16
NIDC = 5

_LANES = 16


def _zero_vmem_rows(ref, nrows, ncols):
    def body(r, _):
        for j in range(ncols // _LANES):
            ref[r, pl.ds(j * _LANES, _LANES)] = jnp.zeros((_LANES,), jnp.float32)
        return 0
    lax.fori_loop(0, nrows, body, 0)


def _edge_aggregate(node_hidden, src1, dst1, edge_hidden):
    mesh = plsc.VectorSubcoreMesh(
        core_axis_name="c", subcore_axis_name="s",
        num_cores=NC, num_subcores=NS)

    @functools.partial(
        pl.kernel,
        out_type=jax.ShapeDtypeStruct((NC, NP, D), jnp.float32),
        mesh=mesh,
        scratch_types=[
            [pltpu.VMEM((CH,), jnp.int32) for _ in range(4)],
            [pltpu.VMEM((CH,), jnp.int32) for _ in range(2)],
            [pltpu.VMEM((CH, D), jnp.float32) for _ in range(2)],
            [pltpu.VMEM((CH, D), jnp.float32) for _ in range(2)],
            pltpu.VMEM((TAIL,), jnp.int32),
            pltpu.VMEM((TAIL,), jnp.int32),
            pltpu.VMEM_SHARED((NP, D), jnp.float32),
            [pltpu.SemaphoreType.DMA for _ in range(4)],
            [pltpu.SemaphoreType.DMA for _ in range(2)],
            [pltpu.SemaphoreType.DMA for _ in range(2)],
            [pltpu.SemaphoreType.DMA for _ in range(2)],
            [pltpu.SemaphoreType.DMA for _ in range(2)],
        ],
    )
    def k(nh_hbm, src_hbm, dst_hbm, eh_hbm, out_hbm,
          sidx, didx, rows, erows, sidx_t, didx_t,
          acc, isem, dsem, gsem, esem, ssem):
        cid = lax.axis_index("c")
        sid = lax.axis_index("s")
        wid = cid * NS + sid

        _zero_vmem_rows(erows[0], CH, D)
        row0 = sid * RPT
        for kk in range(RPT // CH):
            pltpu.sync_copy(erows[0], acc.at[pl.ds(row0 + kk * CH, CH)])
        if RPT % CH:
            pltpu.sync_copy(erows[0].at[pl.ds(0, RPT % CH)],
                            acc.at[pl.ds(row0 + (RPT // CH) * CH, RPT % CH)])
        plsc.subcore_barrier()

        ebase = wid * EPT

        def issue_sidx(c, m):
            pltpu.async_copy(src_hbm.at[pl.ds(ebase + c * CH, CH)],
                             sidx[m], isem[m])

        def wait_sidx(c, m):
            pltpu.make_async_copy(
                src_hbm.at[pl.ds(ebase + c * CH, CH)], sidx[m],
                isem[m]).wait()

        def compute(b, nrows):
            def rbody(r, _):
                for j in range(D // _LANES):
                    sl = pl.ds(j * _LANES, _LANES)
                    v = rows[b][r, sl] + erows[b][r, sl]
                    erows[b][r, sl] = jnp.maximum(v, 0.0)
                return 0
            lax.fori_loop(0, nrows, rbody, 0)

        def body(c, m, b, prefetch_ok, issue_ok, static_tail):
            pltpu.make_async_copy(
                nh_hbm.at[sidx[m]], rows[b], gsem[b]).wait()
            pltpu.make_async_copy(
                eh_hbm.at[pl.ds(ebase + c * CH, CH)], erows[b],
                esem[b]).wait()

            def _prefetch():
                issue_sidx(c + 4, m)
            if prefetch_ok is None:
                pl.when(c + 4 < NF)(_prefetch)
            elif prefetch_ok:
                _prefetch()

            pltpu.async_copy(dst_hbm.at[pl.ds(ebase + c * CH, CH)],
                             didx[b], dsem[b])

            compute(b, CH)

            def _issue_gather():
                wait_sidx(c + 2, (m + 2) % 4)
                pltpu.async_copy(nh_hbm.at[sidx[(m + 2) % 4]], rows[b],
                                 gsem[b])
            if issue_ok is None:
                pl.when(c + 2 < NF)(_issue_gather)
            elif issue_ok:
                _issue_gather()

            pltpu.make_async_copy(
                dst_hbm.at[pl.ds(ebase + c * CH, CH)], didx[b],
                dsem[b]).wait()
            pltpu.async_copy(erows[b], acc.at[didx[b]], ssem[b], add=True)
            pltpu.make_async_copy(
                erows[b], acc.at[didx[b]], ssem[b]).wait()

            def _issue_edge():
                pltpu.async_copy(eh_hbm.at[pl.ds(ebase + (c + 2) * CH, CH)],
                                 erows[b], esem[b])
            if issue_ok is None:
                pl.when(c + 2 < NF)(_issue_edge)
            elif issue_ok:
                _issue_edge()

        for c0 in range(4):
            issue_sidx(c0, c0)
        for c0 in range(2):
            wait_sidx(c0, c0)
            pltpu.async_copy(nh_hbm.at[sidx[c0]], rows[c0], gsem[c0])
            pltpu.async_copy(eh_hbm.at[pl.ds(ebase + c0 * CH, CH)],
                             erows[c0], esem[c0])

        MAIN = (NF - 2) // 4 * 4

        @pl.loop(0, MAIN, step=4)
        def quad(i):
            for q in range(4):
                body(i + q, q, q % 2, None, None, False)

        for c in range(MAIN, NF):
            body(c, c % 4, c % 2, c + 4 < NF, c + 2 < NF, False)

        if TAIL:
            pltpu.sync_copy(src_hbm.at[pl.ds(ebase + NF * CH, TAIL)], sidx_t)
            pltpu.sync_copy(dst_hbm.at[pl.ds(ebase + NF * CH, TAIL)], didx_t)
            pltpu.async_copy(nh_hbm.at[sidx_t], rows[0].at[pl.ds(0, TAIL)],
                             gsem[0]).wait()
            pltpu.sync_copy(eh_hbm.at[pl.ds(ebase + NF * CH, TAIL)],
                            erows[0].at[pl.ds(0, TAIL)])
            compute(0, TAIL)
            pltpu.sync_copy(erows[0].at[pl.ds(0, TAIL)], acc.at[didx_t],
                            add=True)

        plsc.subcore_barrier()

        for kk in range(RPT // RCH):
            r0 = row0 + kk * RCH
            pltpu.sync_copy(acc.at[pl.ds(r0, RCH)],
                            out_hbm.at[cid, pl.ds(r0, RCH)])
        REM = RPT - (RPT // RCH) * RCH
        if REM:
            r0 = row0 + (RPT // RCH) * RCH
            pltpu.sync_copy(acc.at[pl.ds(r0, REM)],
                            out_hbm.at[cid, pl.ds(r0, REM)])

    return k(node_hidden, src1, dst1, edge_hidden)


def _node_kernel(node_hidden, partials, nid2, W1, b1, W2, b2,
                 ln_gamma, ln_beta):
    BN = 1000
    NB = N // BN

    def body(nh_ref, part_ref, nid_ref, w1_ref, b1_ref, w2_ref,
             b2_ref, g_ref, beta_ref, out_ref, cnt_ref):
        ph = pl.program_id(0)
        i = pl.program_id(1)
        ids = nid_ref[...]
        iota = lax.broadcasted_iota(jnp.int32, (BN, NG), 1)
        onehot = (ids == iota).astype(jnp.float32)

        @pl.when((ph == 0) & (i == 0))
        def _init():
            cnt_ref[...] = jnp.zeros_like(cnt_ref)

        @pl.when(ph == 0)
        def _count():
            cnt_ref[...] += jnp.sum(onehot, axis=0, keepdims=True)

        @pl.when(ph == 1)
        def _main():
            nh = nh_ref[...]
            h = nh + part_ref[0] + part_ref[1]
            h1 = jnp.maximum(
                jnp.dot(h, w1_ref[...], preferred_element_type=jnp.float32)
                + b1_ref[...], 0.0)
            h2 = (jnp.dot(h1, w2_ref[...],
                          preferred_element_type=jnp.float32) + b2_ref[...])
            mean = jnp.mean(h2, axis=-1, keepdims=True)
            var = jnp.mean((h2 - mean) ** 2, axis=-1, keepdims=True)
            ln = ((h2 - mean) * lax.rsqrt(var + 1e-5) * g_ref[...]
                  + beta_ref[...])
            cnt = jnp.sum(onehot * cnt_ref[...], axis=-1, keepdims=True)
            out_ref[...] = jnp.maximum(ln * lax.rsqrt(cnt), 0.0) + nh

    return pl.pallas_call(
        body,
        grid=(2, NB),
        in_specs=[
            pl.BlockSpec((BN, D), lambda p, i: (p * i, 0)),
            pl.BlockSpec((NC, BN, D), lambda p, i: (0, p * i, 0)),
            pl.BlockSpec((BN, 1), lambda p, i: (i, 0)),
            pl.BlockSpec((D, 2 * D), lambda p, i: (0, 0)),
            pl.BlockSpec((1, 2 * D), lambda p, i: (0, 0)),
            pl.BlockSpec((2 * D, D), lambda p, i: (0, 0)),
            pl.BlockSpec((1, D), lambda p, i: (0, 0)),
            pl.BlockSpec((1, D), lambda p, i: (0, 0)),
            pl.BlockSpec((1, D), lambda p, i: (0, 0)),
        ],
        out_specs=pl.BlockSpec((BN, D), lambda p, i: (p * i, 0)),
        out_shape=jax.ShapeDtypeStruct((N, D), jnp.float32),
        scratch_shapes=[pltpu.VMEM((1, NG), jnp.float32)],
    )(node_hidden, partials, nid2, W1, b1[None, :], W2, b2[None, :],
      ln_gamma[None, :], ln_beta[None, :])


def kernel(node_hidden, edge_index, edge_hidden, node_id, edge_id,
           W1, b1, W2, b2, ln_gamma, ln_beta):
    src1 = edge_index[0].astype(jnp.int32)
    dst1 = edge_index[1].astype(jnp.int32)
    partials = _edge_aggregate(node_hidden, src1, dst1, edge_hidden)
    nid2 = node_id.astype(jnp.int32).reshape(N, 1)
    return _node_kernel(node_hidden, partials, nid2,
                        W1, b1, W2, b2, ln_gamma, ln_beta)
